# baseline (speedup 1.0000x reference)
import functools

import jax
import jax.numpy as jnp
from jax import lax
from jax.experimental import pallas as pl
from jax.experimental.pallas import tpu as pltpu
from jax.experimental.pallas import tpu_sc as plsc

B = 16384
DIM = 64
WDIM = 128
LANES = 16
NC = 2
NS = 16
NW = NC * NS
BPW = B // NW
NCHUNK_STAGE = 4
CH = BPW // NCHUNK_STAGE
GROUPS_PER_CH = CH // LANES
PROWS = B * DIM // WDIM
PRPW = PROWS // NW
PRPC = PRPW // NCHUNK_STAGE

TC_RB = 2048


def _product_body(eh_ref, et_ref, o_ref):
    p = eh_ref[...] * et_ref[...]
    q = p.reshape(TC_RB // 2, 2, DIM)
    o_ref[...] = jnp.concatenate([q[:, 0, :], q[:, 1, :]], axis=1)


def _product(e_h, e_t):
    grid = (B // TC_RB,)
    return pl.pallas_call(
        _product_body,
        grid=grid,
        in_specs=[
            pl.BlockSpec((TC_RB, DIM), lambda i: (i, 0)),
            pl.BlockSpec((TC_RB, DIM), lambda i: (i, 0)),
        ],
        out_specs=pl.BlockSpec((TC_RB // 2, WDIM), lambda i: (i, 0)),
        out_shape=jax.ShapeDtypeStruct((PROWS, WDIM), jnp.float32),
    )(e_h, e_t)


def _dist_mult_body(p_hbm, r_hbm, w_hbm, out_hbm,
                    idx_v, p_v, w_v, out_v,
                    sem0, sem1, sem2, sem3):
    wid = lax.axis_index("s") * NC + lax.axis_index("c")
    base = wid * BPW
    pbase = wid * PRPW
    sems = [sem0, sem1, sem2, sem3]

    pltpu.sync_copy(r_hbm.at[pl.ds(base, BPW)], idx_v)

    def fire_w(c):
        return pltpu.async_copy(
            w_hbm.at[idx_v.at[pl.ds(c * CH, CH)]],
            w_v.at[c % 2], sems[c])

    chunk_copies = []
    for c in range(NCHUNK_STAGE):
        cps = [
            pltpu.async_copy(
                p_hbm.at[pl.ds(pbase + c * PRPC, PRPC)],
                p_v.at[pl.ds(c * PRPC, PRPC)], sems[c]),
        ]
        if c < 2:
            cps.append(fire_w(c))
        chunk_copies.append(cps)

    lane = lax.iota(jnp.int32, LANES)

    def make_group(wbuf, goff):
        def group(g, carry):
            flat0 = (g * LANES + lane) * DIM
            rowl = (g - goff) * LANES + lane
            accs = [jnp.zeros((LANES,), jnp.float32) for _ in range(4)]
            mask = jnp.full((LANES,), DIM - 1, jnp.int32)
            lomask = jnp.full((LANES,), WDIM - 1, jnp.int32)
            one = jnp.ones((LANES,), jnp.int32)
            col = lane
            for d in range(DIM):
                flat = flat0 + col
                pval = plsc.load_gather(
                    p_v, [lax.shift_right_logical(flat, 7), flat & lomask])
                wval = plsc.load_gather(wbuf, [rowl, col])
                accs[d % 4] = accs[d % 4] + pval * wval
                col = (col + one) & mask
            acc = (accs[0] + accs[1]) + (accs[2] + accs[3])
            out_v[pl.ds(g * LANES, LANES)] = acc
            return carry
        return group

    for c in range(NCHUNK_STAGE):
        for cp in chunk_copies[c]:
            cp.wait()
        lax.fori_loop(c * GROUPS_PER_CH, (c + 1) * GROUPS_PER_CH,
                      make_group(w_v.at[c % 2], c * GROUPS_PER_CH), 0)
        if c + 2 < NCHUNK_STAGE:
            chunk_copies[c + 2].append(fire_w(c + 2))

    pltpu.sync_copy(out_v, out_hbm.at[pl.ds(base, BPW)])


def kernel(e_h, r, e_t, rel_weight):
    p = _product(e_h, e_t)
    w_pad = jnp.pad(rel_weight, ((0, 0), (0, WDIM - DIM)))
    mesh = plsc.VectorSubcoreMesh(core_axis_name="c", subcore_axis_name="s")
    f = pl.kernel(
        _dist_mult_body,
        out_type=jax.ShapeDtypeStruct((B,), jnp.float32),
        mesh=mesh,
        compiler_params=pltpu.CompilerParams(
            needs_layout_passes=False,
            use_tc_tiling_on_sc=True,
        ),
        scratch_types=[
            pltpu.VMEM((BPW,), jnp.int32),
            pltpu.VMEM((PRPW, WDIM), jnp.float32),
            pltpu.VMEM((2, CH, WDIM), jnp.float32),
            pltpu.VMEM((BPW,), jnp.float32),
            pltpu.SemaphoreType.DMA,
            pltpu.SemaphoreType.DMA,
            pltpu.SemaphoreType.DMA,
            pltpu.SemaphoreType.DMA,
        ],
    )
    return f(p, r, w_pad)


# speedup vs baseline: 1.0017x; 1.0017x over previous
import functools

import jax
import jax.numpy as jnp
from jax import lax
from jax.experimental import pallas as pl
from jax.experimental.pallas import tpu as pltpu
from jax.experimental.pallas import tpu_sc as plsc

B = 16384
DIM = 64
WDIM = 128
LANES = 16
NC = 2
NS = 16
NW = NC * NS
BPW = B // NW
NCHUNK_STAGE = 4
CH = BPW // NCHUNK_STAGE
GROUPS_PER_CH = CH // LANES
PROWS = B * DIM // WDIM
PRPW = PROWS // NW
PRPC = PRPW // NCHUNK_STAGE

TC_RB = 2048


def _product_body(eh_ref, et_ref, o_ref):
    p = eh_ref[...] * et_ref[...]
    q = p.reshape(TC_RB // 2, 2, DIM)
    o_ref[...] = jnp.concatenate([q[:, 0, :], q[:, 1, :]], axis=1)


def _product(e_h, e_t):
    grid = (B // TC_RB,)
    return pl.pallas_call(
        _product_body,
        grid=grid,
        in_specs=[
            pl.BlockSpec((TC_RB, DIM), lambda i: (i, 0)),
            pl.BlockSpec((TC_RB, DIM), lambda i: (i, 0)),
        ],
        out_specs=pl.BlockSpec((TC_RB // 2, WDIM), lambda i: (i, 0)),
        out_shape=jax.ShapeDtypeStruct((PROWS, WDIM), jnp.float32),
        compiler_params=pltpu.CompilerParams(
            allow_input_fusion=[True, True],
        ),
    )(e_h, e_t)


def _dist_mult_body(p_hbm, r_hbm, w_hbm, out_hbm,
                    idx_v, p_v, w_v, out_v,
                    sem0, sem1, sem2, sem3):
    wid = lax.axis_index("s") * NC + lax.axis_index("c")
    base = wid * BPW
    pbase = wid * PRPW
    sems = [sem0, sem1, sem2, sem3]

    pltpu.sync_copy(r_hbm.at[pl.ds(base, BPW)], idx_v)

    def fire_w(c):
        return pltpu.async_copy(
            w_hbm.at[idx_v.at[pl.ds(c * CH, CH)]],
            w_v.at[c % 2], sems[c])

    chunk_copies = []
    for c in range(NCHUNK_STAGE):
        cps = [
            pltpu.async_copy(
                p_hbm.at[pl.ds(pbase + c * PRPC, PRPC)],
                p_v.at[pl.ds(c * PRPC, PRPC)], sems[c]),
        ]
        if c < 2:
            cps.append(fire_w(c))
        chunk_copies.append(cps)

    lane = lax.iota(jnp.int32, LANES)

    def make_group(wbuf, goff):
        def group(g, carry):
            flat0 = (g * LANES + lane) * DIM
            rowl = (g - goff) * LANES + lane
            accs = [jnp.zeros((LANES,), jnp.float32) for _ in range(4)]
            mask = jnp.full((LANES,), DIM - 1, jnp.int32)
            lomask = jnp.full((LANES,), WDIM - 1, jnp.int32)
            one = jnp.ones((LANES,), jnp.int32)
            col = lane
            for d in range(DIM):
                flat = flat0 + col
                pval = plsc.load_gather(
                    p_v, [lax.shift_right_logical(flat, 7), flat & lomask])
                wval = plsc.load_gather(wbuf, [rowl, col])
                accs[d % 4] = accs[d % 4] + pval * wval
                col = (col + one) & mask
            acc = (accs[0] + accs[1]) + (accs[2] + accs[3])
            out_v[pl.ds(g * LANES, LANES)] = acc
            return carry
        return group

    for c in range(NCHUNK_STAGE):
        for cp in chunk_copies[c]:
            cp.wait()
        lax.fori_loop(c * GROUPS_PER_CH, (c + 1) * GROUPS_PER_CH,
                      make_group(w_v.at[c % 2], c * GROUPS_PER_CH), 0)
        if c + 2 < NCHUNK_STAGE:
            chunk_copies[c + 2].append(fire_w(c + 2))

    pltpu.sync_copy(out_v, out_hbm.at[pl.ds(base, BPW)])


def kernel(e_h, r, e_t, rel_weight):
    p = _product(e_h, e_t)
    w_pad = jnp.pad(rel_weight, ((0, 0), (0, WDIM - DIM)))
    mesh = plsc.VectorSubcoreMesh(core_axis_name="c", subcore_axis_name="s")
    f = pl.kernel(
        _dist_mult_body,
        out_type=jax.ShapeDtypeStruct((B,), jnp.float32),
        mesh=mesh,
        compiler_params=pltpu.CompilerParams(
            needs_layout_passes=False,
            use_tc_tiling_on_sc=True,
        ),
        scratch_types=[
            pltpu.VMEM((BPW,), jnp.int32),
            pltpu.VMEM((PRPW, WDIM), jnp.float32),
            pltpu.VMEM((2, CH, WDIM), jnp.float32),
            pltpu.VMEM((BPW,), jnp.float32),
            pltpu.SemaphoreType.DMA,
            pltpu.SemaphoreType.DMA,
            pltpu.SemaphoreType.DMA,
            pltpu.SemaphoreType.DMA,
        ],
    )
    return f(p, r, w_pad)


# speedup vs baseline: 1.0291x; 1.0274x over previous
import functools

import jax
import jax.numpy as jnp
from jax import lax
from jax.experimental import pallas as pl
from jax.experimental.pallas import tpu as pltpu
from jax.experimental.pallas import tpu_sc as plsc

B = 16384
DIM = 64
WDIM = 128
LANES = 16
NC = 2
NS = 16
NW = NC * NS
BPW = B // NW
NCHUNK_STAGE = 4
CH = BPW // NCHUNK_STAGE
GROUPS_PER_CH = CH // LANES
PROWS = B * DIM // WDIM
PRPW = PROWS // NW
PRPC = PRPW // NCHUNK_STAGE

TC_RB = 4096


def _product_body(eh_ref, et_ref, o_ref):
    p = eh_ref[...] * et_ref[...]
    q = p.reshape(TC_RB // 2, 2, DIM)
    o_ref[...] = jnp.concatenate([q[:, 0, :], q[:, 1, :]], axis=1)


def _product(e_h, e_t):
    grid = (B // TC_RB,)
    return pl.pallas_call(
        _product_body,
        grid=grid,
        in_specs=[
            pl.BlockSpec((TC_RB, DIM), lambda i: (i, 0)),
            pl.BlockSpec((TC_RB, DIM), lambda i: (i, 0)),
        ],
        out_specs=pl.BlockSpec((TC_RB // 2, WDIM), lambda i: (i, 0)),
        out_shape=jax.ShapeDtypeStruct((PROWS, WDIM), jnp.float32),
        compiler_params=pltpu.CompilerParams(
            allow_input_fusion=[True, True],
        ),
    )(e_h, e_t)


def _dist_mult_body(p_hbm, r_hbm, w_hbm, out_hbm,
                    idx_v, p_v, w_v, out_v,
                    sem0, sem1, sem2, sem3):
    wid = lax.axis_index("s") * NC + lax.axis_index("c")
    base = wid * BPW
    pbase = wid * PRPW
    sems = [sem0, sem1, sem2, sem3]

    pltpu.sync_copy(r_hbm.at[pl.ds(base, BPW)], idx_v)

    def fire_w(c):
        return pltpu.async_copy(
            w_hbm.at[idx_v.at[pl.ds(c * CH, CH)]],
            w_v.at[c % 2], sems[c])

    chunk_copies = []
    for c in range(NCHUNK_STAGE):
        cps = [
            pltpu.async_copy(
                p_hbm.at[pl.ds(pbase + c * PRPC, PRPC)],
                p_v.at[pl.ds(c * PRPC, PRPC)], sems[c]),
        ]
        if c < 2:
            cps.append(fire_w(c))
        chunk_copies.append(cps)

    lane = lax.iota(jnp.int32, LANES)

    def make_group(wbuf, goff):
        def group(g, carry):
            flat0 = (g * LANES + lane) * DIM
            rowl = (g - goff) * LANES + lane
            accs = [jnp.zeros((LANES,), jnp.float32) for _ in range(4)]
            mask = jnp.full((LANES,), DIM - 1, jnp.int32)
            lomask = jnp.full((LANES,), WDIM - 1, jnp.int32)
            one = jnp.ones((LANES,), jnp.int32)
            col = lane
            for d in range(DIM):
                flat = flat0 + col
                pval = plsc.load_gather(
                    p_v, [lax.shift_right_logical(flat, 7), flat & lomask])
                wval = plsc.load_gather(wbuf, [rowl, col])
                accs[d % 4] = accs[d % 4] + pval * wval
                col = (col + one) & mask
            acc = (accs[0] + accs[1]) + (accs[2] + accs[3])
            out_v[pl.ds(g * LANES, LANES)] = acc
            return carry
        return group

    for c in range(NCHUNK_STAGE):
        for cp in chunk_copies[c]:
            cp.wait()
        lax.fori_loop(c * GROUPS_PER_CH, (c + 1) * GROUPS_PER_CH,
                      make_group(w_v.at[c % 2], c * GROUPS_PER_CH), 0)
        if c + 2 < NCHUNK_STAGE:
            chunk_copies[c + 2].append(fire_w(c + 2))

    pltpu.sync_copy(out_v, out_hbm.at[pl.ds(base, BPW)])


def kernel(e_h, r, e_t, rel_weight):
    p = _product(e_h, e_t)
    w_pad = jnp.pad(rel_weight, ((0, 0), (0, WDIM - DIM)))
    mesh = plsc.VectorSubcoreMesh(core_axis_name="c", subcore_axis_name="s")
    f = pl.kernel(
        _dist_mult_body,
        out_type=jax.ShapeDtypeStruct((B,), jnp.float32),
        mesh=mesh,
        compiler_params=pltpu.CompilerParams(
            needs_layout_passes=False,
            use_tc_tiling_on_sc=True,
        ),
        scratch_types=[
            pltpu.VMEM((BPW,), jnp.int32),
            pltpu.VMEM((PRPW, WDIM), jnp.float32),
            pltpu.VMEM((2, CH, WDIM), jnp.float32),
            pltpu.VMEM((BPW,), jnp.float32),
            pltpu.SemaphoreType.DMA,
            pltpu.SemaphoreType.DMA,
            pltpu.SemaphoreType.DMA,
            pltpu.SemaphoreType.DMA,
        ],
    )
    return f(p, r, w_pad)
